# Initial kernel scaffold; baseline (speedup 1.0000x reference)
import jax
import jax.numpy as jnp
from jax.experimental import pallas as pl


def kernel(c, s, c_weight, c_biase, s_weight, s_biase):
    raise NotImplementedError("write your pallas kernel here")


# speedup vs baseline: 9.1702x; 9.1702x over previous
import functools

import jax
import jax.numpy as jnp
from jax import lax
from jax.experimental import pallas as pl
from jax.experimental.pallas import tpu as pltpu
from jax.experimental.pallas import tpu_sc as plsc

_LANES = 16


def _interaction_table_kernel(cw_ref, sw_ref, cb_ref, sb_ref, g_ref):
    g = lax.dot_general(
        cw_ref[...],
        sw_ref[...],
        (((1,), (1,)), ((), ())),
        preferred_element_type=jnp.float32,
        precision=lax.Precision.HIGHEST,
    )
    g_ref[...] = g + cb_ref[...] + sb_ref[...]


def _build_interaction_table(c_weight, s_weight, c_biase, s_biase):
    v = c_weight.shape[0]
    return pl.pallas_call(
        _interaction_table_kernel,
        out_shape=jax.ShapeDtypeStruct((v, v), jnp.float32),
    )(c_weight, s_weight, c_biase, s_biase.reshape(1, v))


def _make_sc_gather(v, b, num_workers, chunk):
    per_w = b // num_workers
    rows = per_w // chunk
    mesh = plsc.VectorSubcoreMesh(core_axis_name="c", subcore_axis_name="s")

    @functools.partial(
        pl.kernel,
        mesh=mesh,
        out_type=jax.ShapeDtypeStruct((num_workers, rows, chunk), jnp.float32),
        scratch_types=[
            pltpu.VMEM((rows, chunk), jnp.int32),
            pltpu.VMEM((rows, chunk), jnp.int32),
            pltpu.VMEM((rows, chunk), jnp.int32),
            pltpu.VMEM((rows, chunk), jnp.float32),
            pltpu.SemaphoreType.DMA,
        ],
    )
    def sc_gather(g_hbm, c_hbm, s_hbm, out_hbm, c_v, s_v, idx_v, val_v, sem):
        wid = lax.axis_index("s") * 2 + lax.axis_index("c")
        pltpu.sync_copy(c_hbm.at[wid], c_v)
        pltpu.sync_copy(s_hbm.at[wid], s_v)
        for r in range(rows):
            for i in range(chunk // _LANES):
                sl = pl.ds(i * _LANES, _LANES)
                idx_v[r, sl] = c_v[r, sl] * v + s_v[r, sl]
        copies = [
            pltpu.async_copy(g_hbm.at[idx_v.at[r]], val_v.at[r], sem)
            for r in range(rows)
        ]
        for cp in copies:
            cp.wait()
        pltpu.sync_copy(val_v, out_hbm.at[wid])

    return sc_gather


def kernel(c, s, c_weight, c_biase, s_weight, s_biase):
    v, _ = c_weight.shape
    b = c.shape[0]
    num_workers = 32
    chunk = 128
    per_w = b // num_workers
    rows = per_w // chunk

    g = _build_interaction_table(c_weight, s_weight, c_biase, s_biase)
    g_flat = g.reshape(v * v)

    c3 = c.astype(jnp.int32).reshape(num_workers, rows, chunk)
    s3 = s.astype(jnp.int32).reshape(num_workers, rows, chunk)

    out = _make_sc_gather(v, b, num_workers, chunk)(g_flat, c3, s3)
    return out.reshape(b, 1)
